# baseline (speedup 1.0000x reference)
import jax
import jax.numpy as jnp
from jax.experimental import pallas as pl

_DEPTH = 1000
_BB = 32


def _onehot_block(idx_ref, out_ref):
    idx = idx_ref[...]
    iota = jax.lax.broadcasted_iota(jnp.int32, out_ref.shape, 2)
    out_ref[...] = jnp.where(idx[..., None] == iota, jnp.float32(1.0),
                             jnp.float32(0.0))


def kernel(indices):
    B, P = indices.shape
    indices = indices.astype(jnp.int32)
    return pl.pallas_call(
        _onehot_block,
        grid=(B // _BB,),
        in_specs=[pl.BlockSpec((_BB, P), lambda i: (i, 0))],
        out_specs=pl.BlockSpec((_BB, P, _DEPTH), lambda i: (i, 0, 0)),
        out_shape=jax.ShapeDtypeStruct((B, P, _DEPTH), jnp.float32),
    )(indices)
